# baseline (speedup 1.0000x reference)
import jax
import jax.numpy as jnp
from jax import lax
from jax.experimental import pallas as pl
from jax.experimental.pallas import tpu as pltpu
from jax.experimental.pallas import tpu_sc as plsc

N_NODES = 10000
D_FEAT = 128
HIDDEN = 64
N_CLASSES = 40
C_PAD = 48

NC, NS = 2, 16
NW = NC * NS
B = 128
NBUF = 4
TRASH = N_NODES
NPAD = 10112
RPT = NPAD // NS
ROWBLK = 1000


def _make_sc_agg(width, with_cnt, g_ops):
    mesh = plsc.VectorSubcoreMesh(core_axis_name="c", subcore_axis_name="s")
    acc_type = jax.ShapeDtypeStruct((NC, NPAD, width), jnp.float32)
    out_type = [acc_type]
    scratch = [
        pltpu.VMEM((g_ops, B), jnp.int32),
        pltpu.VMEM((g_ops, B), jnp.int32),
        pltpu.VMEM((NBUF, B, width), jnp.float32),
        pltpu.VMEM_SHARED((NPAD, width), jnp.float32),
        pltpu.SemaphoreType.DMA((NBUF,)),
        pltpu.SemaphoreType.DMA((NBUF,)),
    ]
    if with_cnt:
        out_type.append(jax.ShapeDtypeStruct((NC, NPAD, 8), jnp.float32))
        scratch += [
            pltpu.VMEM((B, 8), jnp.float32),
            pltpu.VMEM_SHARED((NPAD, 8), jnp.float32),
            pltpu.SemaphoreType.DMA,
        ]

    def body(src_hbm, dst_hbm, y_hbm, zw_hbm, *rest):
        if with_cnt:
            (z8_hbm, ones_hbm, acc_out, cnt_out,
             src_v, dst_v, rows_v, acc_sh, gsem, ssem,
             ones_v, cnt_sh, csem) = rest
        else:
            (acc_out, src_v, dst_v, rows_v, acc_sh, gsem, ssem) = rest
        c = lax.axis_index("c")
        s = lax.axis_index("s")
        wid = c * NS + s
        pltpu.sync_copy(src_hbm.at[wid], src_v)
        pltpu.sync_copy(dst_hbm.at[wid], dst_v)
        pltpu.sync_copy(zw_hbm, acc_sh.at[pl.ds(s * RPT, RPT)])
        if with_cnt:
            pltpu.sync_copy(ones_hbm, ones_v)
            pltpu.sync_copy(z8_hbm, cnt_sh.at[pl.ds(s * RPT, RPT)])
        plsc.subcore_barrier()

        def issue_gather(i, b):
            pltpu.async_copy(y_hbm.at[src_v.at[i]], rows_v.at[b], gsem.at[b])

        def wait_gather(b):
            pltpu.make_async_copy(y_hbm.at[src_v.at[0]], rows_v.at[b],
                                  gsem.at[b]).wait()

        def issue_scatter(i, b):
            pltpu.async_copy(rows_v.at[b], acc_sh.at[dst_v.at[i]],
                             ssem.at[b], add=True)
            if with_cnt:
                pltpu.async_copy(ones_v, cnt_sh.at[dst_v.at[i]], csem,
                                 add=True)

        def wait_scatter(b):
            pltpu.make_async_copy(rows_v.at[b], acc_sh.at[dst_v.at[0]],
                                  ssem.at[b]).wait()

        def wait_cnt():
            pltpu.make_async_copy(ones_v, cnt_sh.at[dst_v.at[0]],
                                  csem).wait()

        issue_gather(0, 0)
        issue_gather(1, 1)
        for b in range(NBUF):
            if b < 2:
                issue_gather(b + 2, (b + 2) % NBUF)
            else:
                wait_scatter((b + 2) % NBUF)
                issue_gather(b + 2, (b + 2) % NBUF)
                if with_cnt:
                    wait_cnt()
            wait_gather(b)
            issue_scatter(b, b)

        @pl.loop(NBUF, g_ops - NBUF, step=NBUF)
        def _blk(blk):
            for b in range(NBUF):
                i = blk + b
                nb = (b + 2) % NBUF
                wait_scatter(nb)
                issue_gather(i + 2, nb)
                if with_cnt:
                    wait_cnt()
                wait_gather(b)
                issue_scatter(i, b)

        for b in range(NBUF):
            i = g_ops - NBUF + b
            if b < 2:
                wait_scatter((b + 2) % NBUF)
                issue_gather(i + 2, (b + 2) % NBUF)
                if with_cnt:
                    wait_cnt()
            wait_gather(b)
            issue_scatter(i, b)
        for b in range(NBUF):
            wait_scatter(b)
        if with_cnt:
            for _ in range(NBUF - 2):
                wait_cnt()
            wait_cnt()
            wait_cnt()

        plsc.subcore_barrier()
        row0 = s * RPT
        pltpu.sync_copy(acc_sh.at[pl.ds(row0, RPT)],
                        acc_out.at[c, pl.ds(row0, RPT)])
        if with_cnt:
            pltpu.sync_copy(cnt_sh.at[pl.ds(row0, RPT)],
                            cnt_out.at[c, pl.ds(row0, RPT)])

    return pl.kernel(body, out_type=out_type if with_cnt else acc_type,
                     mesh=mesh, scratch_types=scratch,
                     compiler_params=pltpu.CompilerParams(
                         use_tc_tiling_on_sc=False))


def _t1(x_ref, w_ref, y1_ref, r1_ref):
    o = jnp.dot(x_ref[...], w_ref[...], preferred_element_type=jnp.float32)
    y1_ref[...] = o[:, :HIDDEN]
    r1_ref[...] = o[:, HIDDEN:]


def _t2(acc_ref, cnt_ref, r1_ref, b1_ref, w2l_ref, w2r_ref, b2_ref,
        y2_ref, r2_ref):
    agg = acc_ref[0] + acc_ref[1]
    cnt = jnp.maximum(cnt_ref[0, :, 0:1] + cnt_ref[1, :, 0:1], 1.0)
    h = jnp.maximum(agg / cnt + b1_ref[...] + r1_ref[...], 0.0)
    y2_ref[...] = jnp.dot(h, w2l_ref[...], preferred_element_type=jnp.float32)
    r2_ref[...] = (jnp.dot(h, w2r_ref[...], preferred_element_type=jnp.float32)
                   + b2_ref[...])


def _t3(acc2_ref, cnt_ref, r2_ref, out_ref):
    agg = acc2_ref[0] + acc2_ref[1]
    cnt = jnp.maximum(cnt_ref[0, :, 0:1] + cnt_ref[1, :, 0:1], 1.0)
    logits = agg / cnt + r2_ref[...]
    m = jnp.max(logits, axis=1, keepdims=True)
    s = jnp.sum(jnp.exp(logits - m), axis=1, keepdims=True)
    out = logits - m - jnp.log(s)
    out_ref[...] = out[:, :N_CLASSES]


_GRID = (N_NODES // ROWBLK,)

_t1_call = pl.pallas_call(
    _t1,
    grid=_GRID,
    in_specs=[
        pl.BlockSpec((ROWBLK, D_FEAT), lambda i: (i, 0)),
        pl.BlockSpec((D_FEAT, 2 * HIDDEN), lambda i: (0, 0)),
    ],
    out_specs=[
        pl.BlockSpec((ROWBLK, HIDDEN), lambda i: (i, 0)),
        pl.BlockSpec((ROWBLK, HIDDEN), lambda i: (i, 0)),
    ],
    out_shape=[jax.ShapeDtypeStruct((N_NODES, HIDDEN), jnp.float32)] * 2,
)

_t2_call = pl.pallas_call(
    _t2,
    grid=_GRID,
    in_specs=[
        pl.BlockSpec((NC, ROWBLK, HIDDEN), lambda i: (0, i, 0)),
        pl.BlockSpec((NC, ROWBLK, 8), lambda i: (0, i, 0)),
        pl.BlockSpec((ROWBLK, HIDDEN), lambda i: (i, 0)),
        pl.BlockSpec((1, HIDDEN), lambda i: (0, 0)),
        pl.BlockSpec((HIDDEN, C_PAD), lambda i: (0, 0)),
        pl.BlockSpec((HIDDEN, C_PAD), lambda i: (0, 0)),
        pl.BlockSpec((1, C_PAD), lambda i: (0, 0)),
    ],
    out_specs=[
        pl.BlockSpec((ROWBLK, C_PAD), lambda i: (i, 0)),
        pl.BlockSpec((ROWBLK, C_PAD), lambda i: (i, 0)),
    ],
    out_shape=[jax.ShapeDtypeStruct((N_NODES, C_PAD), jnp.float32)] * 2,
)

_t3_call = pl.pallas_call(
    _t3,
    grid=_GRID,
    in_specs=[
        pl.BlockSpec((NC, ROWBLK, C_PAD), lambda i: (0, i, 0)),
        pl.BlockSpec((NC, ROWBLK, 8), lambda i: (0, i, 0)),
        pl.BlockSpec((ROWBLK, C_PAD), lambda i: (i, 0)),
    ],
    out_specs=pl.BlockSpec((ROWBLK, N_CLASSES), lambda i: (i, 0)),
    out_shape=jax.ShapeDtypeStruct((N_NODES, N_CLASSES), jnp.float32),
)


def kernel(x, edge_index, W1l, b1, W1r, W2l, b2, W2r):
    e = edge_index.shape[1]
    g_ops = -(-e // (NW * B))
    g_ops = -(-g_ops // NBUF) * NBUF
    e_pad = NW * B * g_ops
    src = edge_index[0].astype(jnp.int32)
    dst = edge_index[1].astype(jnp.int32)
    src = jnp.concatenate([src, jnp.zeros((e_pad - e,), jnp.int32)])
    dst = jnp.concatenate([dst, jnp.full((e_pad - e,), TRASH, jnp.int32)])
    src3 = src.reshape(NW, g_ops, B)
    dst3 = dst.reshape(NW, g_ops, B)

    zeros_h = jnp.zeros((RPT, HIDDEN), jnp.float32)
    zeros_c = jnp.zeros((RPT, C_PAD), jnp.float32)
    zeros_8 = jnp.zeros((RPT, 8), jnp.float32)
    ones_8 = jnp.ones((B, 8), jnp.float32)

    w1cat = jnp.concatenate([W1l.T, W1r.T], axis=1)
    w2l_t = jnp.zeros((HIDDEN, C_PAD), jnp.float32).at[:, :N_CLASSES].set(W2l.T)
    w2r_t = jnp.zeros((HIDDEN, C_PAD), jnp.float32).at[:, :N_CLASSES].set(W2r.T)
    b2_pad = jnp.full((1, C_PAD), -1e30, jnp.float32).at[0, :N_CLASSES].set(b2)
    b1_row = b1.reshape(1, HIDDEN)

    y1, r1 = _t1_call(x, w1cat)
    acc1, cnt = _make_sc_agg(HIDDEN, True, g_ops)(
        src3, dst3, y1, zeros_h, zeros_8, ones_8)
    y2, r2 = _t2_call(acc1, cnt, r1, b1_row, w2l_t, w2r_t, b2_pad)
    acc2 = _make_sc_agg(C_PAD, False, g_ops)(src3, dst3, y2, zeros_c)
    return _t3_call(acc2, cnt, r2)


# speedup vs baseline: 1.0013x; 1.0013x over previous
import jax
import jax.numpy as jnp
from jax import lax
from jax.experimental import pallas as pl
from jax.experimental.pallas import tpu as pltpu
from jax.experimental.pallas import tpu_sc as plsc

N_NODES = 10000
D_FEAT = 128
HIDDEN = 64
N_CLASSES = 40
C_PAD = 48

NC, NS = 2, 16
NW = NC * NS
B = 128
NBUF = 4
DEPTH = 3
TRASH = N_NODES
NPAD = 10112
RPT = NPAD // NS
ROWBLK = 1000


def _make_sc_agg(width, with_cnt, g_ops):
    mesh = plsc.VectorSubcoreMesh(core_axis_name="c", subcore_axis_name="s")
    acc_type = jax.ShapeDtypeStruct((NC, NPAD, width), jnp.float32)
    out_type = [acc_type]
    scratch = [
        pltpu.VMEM((g_ops, B), jnp.int32),
        pltpu.VMEM((g_ops, B), jnp.int32),
        pltpu.VMEM((NBUF, B, width), jnp.float32),
        pltpu.VMEM_SHARED((NPAD, width), jnp.float32),
        pltpu.SemaphoreType.DMA((NBUF,)),
        pltpu.SemaphoreType.DMA((NBUF,)),
    ]
    if with_cnt:
        out_type.append(jax.ShapeDtypeStruct((NC, NPAD, 8), jnp.float32))
        scratch += [
            pltpu.VMEM((B, 8), jnp.float32),
            pltpu.VMEM_SHARED((NPAD, 8), jnp.float32),
            pltpu.SemaphoreType.DMA,
        ]

    def body(src_hbm, dst_hbm, y_hbm, zw_hbm, *rest):
        if with_cnt:
            (z8_hbm, ones_hbm, acc_out, cnt_out,
             src_v, dst_v, rows_v, acc_sh, gsem, ssem,
             ones_v, cnt_sh, csem) = rest
        else:
            (acc_out, src_v, dst_v, rows_v, acc_sh, gsem, ssem) = rest
        c = lax.axis_index("c")
        s = lax.axis_index("s")
        wid = c * NS + s
        pltpu.sync_copy(src_hbm.at[wid], src_v)
        pltpu.sync_copy(dst_hbm.at[wid], dst_v)
        pltpu.sync_copy(zw_hbm, acc_sh.at[pl.ds(s * RPT, RPT)])
        if with_cnt:
            pltpu.sync_copy(ones_hbm, ones_v)
            pltpu.sync_copy(z8_hbm, cnt_sh.at[pl.ds(s * RPT, RPT)])
        plsc.subcore_barrier()

        def issue_gather(i, b):
            pltpu.async_copy(y_hbm.at[src_v.at[i]], rows_v.at[b], gsem.at[b])

        def wait_gather(b):
            pltpu.make_async_copy(y_hbm.at[src_v.at[0]], rows_v.at[b],
                                  gsem.at[b]).wait()

        def issue_scatter(i, b):
            pltpu.async_copy(rows_v.at[b], acc_sh.at[dst_v.at[i]],
                             ssem.at[b], add=True)
            if with_cnt:
                pltpu.async_copy(ones_v, cnt_sh.at[dst_v.at[i]], csem,
                                 add=True)

        def wait_scatter(b):
            pltpu.make_async_copy(rows_v.at[b], acc_sh.at[dst_v.at[0]],
                                  ssem.at[b]).wait()

        def wait_cnt():
            pltpu.make_async_copy(ones_v, cnt_sh.at[dst_v.at[0]],
                                  csem).wait()

        for i in range(DEPTH):
            issue_gather(i, i)
        for b in range(NBUF):
            nb = (b + DEPTH) % NBUF
            if b >= NBUF - DEPTH:
                wait_scatter(nb)
                if with_cnt:
                    wait_cnt()
            issue_gather(b + DEPTH, nb)
            wait_gather(b)
            issue_scatter(b, b)

        @pl.loop(NBUF, g_ops - NBUF, step=NBUF)
        def _blk(blk):
            for b in range(NBUF):
                i = blk + b
                nb = (b + DEPTH) % NBUF
                wait_scatter(nb)
                issue_gather(i + DEPTH, nb)
                if with_cnt:
                    wait_cnt()
                wait_gather(b)
                issue_scatter(i, b)

        for b in range(NBUF):
            i = g_ops - NBUF + b
            if b < NBUF - DEPTH:
                wait_scatter((b + DEPTH) % NBUF)
                issue_gather(i + DEPTH, (b + DEPTH) % NBUF)
                if with_cnt:
                    wait_cnt()
            wait_gather(b)
            issue_scatter(i, b)
        for b in range(NBUF):
            wait_scatter(b)
        if with_cnt:
            for _ in range(NBUF):
                wait_cnt()

        plsc.subcore_barrier()
        row0 = s * RPT
        pltpu.sync_copy(acc_sh.at[pl.ds(row0, RPT)],
                        acc_out.at[c, pl.ds(row0, RPT)])
        if with_cnt:
            pltpu.sync_copy(cnt_sh.at[pl.ds(row0, RPT)],
                            cnt_out.at[c, pl.ds(row0, RPT)])

    return pl.kernel(body, out_type=out_type if with_cnt else acc_type,
                     mesh=mesh, scratch_types=scratch,
                     compiler_params=pltpu.CompilerParams(
                         use_tc_tiling_on_sc=False))


def _t1(x_ref, w_ref, y1_ref, r1_ref):
    o = jnp.dot(x_ref[...], w_ref[...], preferred_element_type=jnp.float32)
    y1_ref[...] = o[:, :HIDDEN]
    r1_ref[...] = o[:, HIDDEN:]


def _t2(acc_ref, cnt_ref, r1_ref, b1_ref, w2l_ref, w2r_ref, b2_ref,
        y2_ref, r2_ref):
    agg = acc_ref[0] + acc_ref[1]
    cnt = jnp.maximum(cnt_ref[0, :, 0:1] + cnt_ref[1, :, 0:1], 1.0)
    h = jnp.maximum(agg / cnt + b1_ref[...] + r1_ref[...], 0.0)
    y2_ref[...] = jnp.dot(h, w2l_ref[...], preferred_element_type=jnp.float32)
    r2_ref[...] = (jnp.dot(h, w2r_ref[...], preferred_element_type=jnp.float32)
                   + b2_ref[...])


def _t3(acc2_ref, cnt_ref, r2_ref, out_ref):
    agg = acc2_ref[0] + acc2_ref[1]
    cnt = jnp.maximum(cnt_ref[0, :, 0:1] + cnt_ref[1, :, 0:1], 1.0)
    logits = agg / cnt + r2_ref[...]
    m = jnp.max(logits, axis=1, keepdims=True)
    s = jnp.sum(jnp.exp(logits - m), axis=1, keepdims=True)
    out = logits - m - jnp.log(s)
    out_ref[...] = out[:, :N_CLASSES]


_GRID = (N_NODES // ROWBLK,)

_t1_call = pl.pallas_call(
    _t1,
    grid=_GRID,
    in_specs=[
        pl.BlockSpec((ROWBLK, D_FEAT), lambda i: (i, 0)),
        pl.BlockSpec((D_FEAT, 2 * HIDDEN), lambda i: (0, 0)),
    ],
    out_specs=[
        pl.BlockSpec((ROWBLK, HIDDEN), lambda i: (i, 0)),
        pl.BlockSpec((ROWBLK, HIDDEN), lambda i: (i, 0)),
    ],
    out_shape=[jax.ShapeDtypeStruct((N_NODES, HIDDEN), jnp.float32)] * 2,
)

_t2_call = pl.pallas_call(
    _t2,
    grid=_GRID,
    in_specs=[
        pl.BlockSpec((NC, ROWBLK, HIDDEN), lambda i: (0, i, 0)),
        pl.BlockSpec((NC, ROWBLK, 8), lambda i: (0, i, 0)),
        pl.BlockSpec((ROWBLK, HIDDEN), lambda i: (i, 0)),
        pl.BlockSpec((1, HIDDEN), lambda i: (0, 0)),
        pl.BlockSpec((HIDDEN, C_PAD), lambda i: (0, 0)),
        pl.BlockSpec((HIDDEN, C_PAD), lambda i: (0, 0)),
        pl.BlockSpec((1, C_PAD), lambda i: (0, 0)),
    ],
    out_specs=[
        pl.BlockSpec((ROWBLK, C_PAD), lambda i: (i, 0)),
        pl.BlockSpec((ROWBLK, C_PAD), lambda i: (i, 0)),
    ],
    out_shape=[jax.ShapeDtypeStruct((N_NODES, C_PAD), jnp.float32)] * 2,
)

_t3_call = pl.pallas_call(
    _t3,
    grid=_GRID,
    in_specs=[
        pl.BlockSpec((NC, ROWBLK, C_PAD), lambda i: (0, i, 0)),
        pl.BlockSpec((NC, ROWBLK, 8), lambda i: (0, i, 0)),
        pl.BlockSpec((ROWBLK, C_PAD), lambda i: (i, 0)),
    ],
    out_specs=pl.BlockSpec((ROWBLK, N_CLASSES), lambda i: (i, 0)),
    out_shape=jax.ShapeDtypeStruct((N_NODES, N_CLASSES), jnp.float32),
)


def kernel(x, edge_index, W1l, b1, W1r, W2l, b2, W2r):
    e = edge_index.shape[1]
    g_ops = -(-e // (NW * B))
    g_ops = -(-g_ops // NBUF) * NBUF
    e_pad = NW * B * g_ops
    src = edge_index[0].astype(jnp.int32)
    dst = edge_index[1].astype(jnp.int32)
    src = jnp.concatenate([src, jnp.zeros((e_pad - e,), jnp.int32)])
    dst = jnp.concatenate([dst, jnp.full((e_pad - e,), TRASH, jnp.int32)])
    src3 = src.reshape(NW, g_ops, B)
    dst3 = dst.reshape(NW, g_ops, B)

    zeros_h = jnp.zeros((RPT, HIDDEN), jnp.float32)
    zeros_c = jnp.zeros((RPT, C_PAD), jnp.float32)
    zeros_8 = jnp.zeros((RPT, 8), jnp.float32)
    ones_8 = jnp.ones((B, 8), jnp.float32)

    w1cat = jnp.concatenate([W1l.T, W1r.T], axis=1)
    w2l_t = jnp.zeros((HIDDEN, C_PAD), jnp.float32).at[:, :N_CLASSES].set(W2l.T)
    w2r_t = jnp.zeros((HIDDEN, C_PAD), jnp.float32).at[:, :N_CLASSES].set(W2r.T)
    b2_pad = jnp.full((1, C_PAD), -1e30, jnp.float32).at[0, :N_CLASSES].set(b2)
    b1_row = b1.reshape(1, HIDDEN)

    y1, r1 = _t1_call(x, w1cat)
    acc1, cnt = _make_sc_agg(HIDDEN, True, g_ops)(
        src3, dst3, y1, zeros_h, zeros_8, ones_8)
    y2, r2 = _t2_call(acc1, cnt, r1, b1_row, w2l_t, w2r_t, b2_pad)
    acc2 = _make_sc_agg(C_PAD, False, g_ops)(src3, dst3, y2, zeros_c)
    return _t3_call(acc2, cnt, r2)


# speedup vs baseline: 1.6882x; 1.6859x over previous
import jax
import jax.numpy as jnp
from jax import lax
from jax.experimental import pallas as pl
from jax.experimental.pallas import tpu as pltpu
from jax.experimental.pallas import tpu_sc as plsc

N_NODES = 10000
D_FEAT = 128
HIDDEN = 64
N_CLASSES = 40
C_PAD = 48

NC, NS = 2, 16
NW = NC * NS
B = 128
NBUF = 4
DEPTH = 3
TRASH = N_NODES
NPAD = 10112
RPT = NPAD // NS
ROWBLK = 1000


def _make_sc_agg(width, with_cnt, g_ops):
    mesh = plsc.VectorSubcoreMesh(core_axis_name="c", subcore_axis_name="s")
    acc_type = jax.ShapeDtypeStruct((NC, NPAD, width), jnp.float32)
    out_type = [acc_type]
    scratch = [
        pltpu.VMEM((g_ops, B), jnp.int32),
        pltpu.VMEM((g_ops, B), jnp.int32),
        pltpu.VMEM((B, width), jnp.float32),
        pltpu.VMEM_SHARED((NPAD, width), jnp.float32),
        pltpu.VMEM_SHARED((NPAD, width), jnp.float32),
        pltpu.SemaphoreType.DMA,
    ]
    if with_cnt:
        out_type.append(jax.ShapeDtypeStruct((NC, NPAD, 8), jnp.float32))
        scratch += [
            pltpu.VMEM((B, 8), jnp.float32),
            pltpu.VMEM_SHARED((NPAD, 8), jnp.float32),
        ]

    def body(src_hbm, dst_hbm, y_hbm, zw_hbm, *rest):
        if with_cnt:
            (z8_hbm, ones_hbm, acc_out, cnt_out,
             src_v, dst_v, rows_v, y_sh, acc_sh, gsem, ones_v, cnt_sh) = rest
        else:
            (acc_out, src_v, dst_v, rows_v, y_sh, acc_sh, gsem) = rest
        c = lax.axis_index("c")
        s = lax.axis_index("s")
        wid = c * NS + s
        row0 = s * RPT
        pltpu.sync_copy(src_hbm.at[wid], src_v)
        pltpu.sync_copy(dst_hbm.at[wid], dst_v)
        pltpu.sync_copy(y_hbm.at[pl.ds(row0, RPT)], y_sh.at[pl.ds(row0, RPT)])
        pltpu.sync_copy(zw_hbm, acc_sh.at[pl.ds(row0, RPT)])
        if with_cnt:
            pltpu.sync_copy(ones_hbm, ones_v)
            pltpu.sync_copy(z8_hbm, cnt_sh.at[pl.ds(row0, RPT)])
        plsc.subcore_barrier()

        def step(g, carry):
            pltpu.async_copy(y_sh.at[src_v.at[g]], rows_v, gsem).wait()
            pltpu.sync_copy(rows_v, acc_sh.at[dst_v.at[g]], add=True)
            if with_cnt:
                pltpu.sync_copy(ones_v, cnt_sh.at[dst_v.at[g]], add=True)
            return carry

        lax.fori_loop(0, g_ops, step, 0)
        plsc.subcore_barrier()
        row0 = s * RPT
        pltpu.sync_copy(acc_sh.at[pl.ds(row0, RPT)],
                        acc_out.at[c, pl.ds(row0, RPT)])
        if with_cnt:
            pltpu.sync_copy(cnt_sh.at[pl.ds(row0, RPT)],
                            cnt_out.at[c, pl.ds(row0, RPT)])

    return pl.kernel(body, out_type=out_type if with_cnt else acc_type,
                     mesh=mesh, scratch_types=scratch,
                     compiler_params=pltpu.CompilerParams(
                         use_tc_tiling_on_sc=False))


def _t1(x_ref, w_ref, y1_ref, r1_ref):
    o = jnp.dot(x_ref[...], w_ref[...], preferred_element_type=jnp.float32)
    y1_ref[...] = o[:, :HIDDEN]
    r1_ref[...] = o[:, HIDDEN:]


def _t2(acc_ref, cnt_ref, r1_ref, b1_ref, w2l_ref, w2r_ref, b2_ref,
        y2_ref, r2_ref):
    agg = acc_ref[0] + acc_ref[1]
    cnt = jnp.maximum(cnt_ref[0, :, 0:1] + cnt_ref[1, :, 0:1], 1.0)
    h = jnp.maximum(agg / cnt + b1_ref[...] + r1_ref[...], 0.0)
    y2_ref[...] = jnp.dot(h, w2l_ref[...], preferred_element_type=jnp.float32)
    r2_ref[...] = (jnp.dot(h, w2r_ref[...], preferred_element_type=jnp.float32)
                   + b2_ref[...])


def _t3(acc2_ref, cnt_ref, r2_ref, out_ref):
    agg = acc2_ref[0] + acc2_ref[1]
    cnt = jnp.maximum(cnt_ref[0, :, 0:1] + cnt_ref[1, :, 0:1], 1.0)
    logits = agg / cnt + r2_ref[...]
    m = jnp.max(logits, axis=1, keepdims=True)
    s = jnp.sum(jnp.exp(logits - m), axis=1, keepdims=True)
    out = logits - m - jnp.log(s)
    out_ref[...] = out[:, :N_CLASSES]


_GRID = (N_NODES // ROWBLK,)

_t1_call = pl.pallas_call(
    _t1,
    grid=_GRID,
    in_specs=[
        pl.BlockSpec((ROWBLK, D_FEAT), lambda i: (i, 0)),
        pl.BlockSpec((D_FEAT, 2 * HIDDEN), lambda i: (0, 0)),
    ],
    out_specs=[
        pl.BlockSpec((ROWBLK, HIDDEN), lambda i: (i, 0)),
        pl.BlockSpec((ROWBLK, HIDDEN), lambda i: (i, 0)),
    ],
    out_shape=[jax.ShapeDtypeStruct((NPAD, HIDDEN), jnp.float32),
               jax.ShapeDtypeStruct((N_NODES, HIDDEN), jnp.float32)],
)

_t2_call = pl.pallas_call(
    _t2,
    grid=_GRID,
    in_specs=[
        pl.BlockSpec((NC, ROWBLK, HIDDEN), lambda i: (0, i, 0)),
        pl.BlockSpec((NC, ROWBLK, 8), lambda i: (0, i, 0)),
        pl.BlockSpec((ROWBLK, HIDDEN), lambda i: (i, 0)),
        pl.BlockSpec((1, HIDDEN), lambda i: (0, 0)),
        pl.BlockSpec((HIDDEN, C_PAD), lambda i: (0, 0)),
        pl.BlockSpec((HIDDEN, C_PAD), lambda i: (0, 0)),
        pl.BlockSpec((1, C_PAD), lambda i: (0, 0)),
    ],
    out_specs=[
        pl.BlockSpec((ROWBLK, C_PAD), lambda i: (i, 0)),
        pl.BlockSpec((ROWBLK, C_PAD), lambda i: (i, 0)),
    ],
    out_shape=[jax.ShapeDtypeStruct((NPAD, C_PAD), jnp.float32),
               jax.ShapeDtypeStruct((N_NODES, C_PAD), jnp.float32)],
)

_t3_call = pl.pallas_call(
    _t3,
    grid=_GRID,
    in_specs=[
        pl.BlockSpec((NC, ROWBLK, C_PAD), lambda i: (0, i, 0)),
        pl.BlockSpec((NC, ROWBLK, 8), lambda i: (0, i, 0)),
        pl.BlockSpec((ROWBLK, C_PAD), lambda i: (i, 0)),
    ],
    out_specs=pl.BlockSpec((ROWBLK, N_CLASSES), lambda i: (i, 0)),
    out_shape=jax.ShapeDtypeStruct((N_NODES, N_CLASSES), jnp.float32),
)


def kernel(x, edge_index, W1l, b1, W1r, W2l, b2, W2r):
    e = edge_index.shape[1]
    g_ops = -(-e // (NW * B))
    g_ops = -(-g_ops // NBUF) * NBUF
    e_pad = NW * B * g_ops
    src = edge_index[0].astype(jnp.int32)
    dst = edge_index[1].astype(jnp.int32)
    src = jnp.concatenate([src, jnp.zeros((e_pad - e,), jnp.int32)])
    dst = jnp.concatenate([dst, jnp.full((e_pad - e,), TRASH, jnp.int32)])
    src3 = src.reshape(NW, g_ops, B)
    dst3 = dst.reshape(NW, g_ops, B)

    zeros_h = jnp.zeros((RPT, HIDDEN), jnp.float32)
    zeros_c = jnp.zeros((RPT, C_PAD), jnp.float32)
    zeros_8 = jnp.zeros((RPT, 8), jnp.float32)
    ones_8 = jnp.ones((B, 8), jnp.float32)

    w1cat = jnp.concatenate([W1l.T, W1r.T], axis=1)
    w2l_t = jnp.zeros((HIDDEN, C_PAD), jnp.float32).at[:, :N_CLASSES].set(W2l.T)
    w2r_t = jnp.zeros((HIDDEN, C_PAD), jnp.float32).at[:, :N_CLASSES].set(W2r.T)
    b2_pad = jnp.full((1, C_PAD), -1e30, jnp.float32).at[0, :N_CLASSES].set(b2)
    b1_row = b1.reshape(1, HIDDEN)

    y1, r1 = _t1_call(x, w1cat)
    acc1, cnt = _make_sc_agg(HIDDEN, True, g_ops)(
        src3, dst3, y1, zeros_h, zeros_8, ones_8)
    y2, r2 = _t2_call(acc1, cnt, r1, b1_row, w2l_t, w2r_t, b2_pad)
    acc2 = _make_sc_agg(C_PAD, False, g_ops)(src3, dst3, y2, zeros_c)
    return _t3_call(acc2, cnt, r2)


# speedup vs baseline: 1.8337x; 1.0862x over previous
import jax
import jax.numpy as jnp
from jax import lax
from jax.experimental import pallas as pl
from jax.experimental.pallas import tpu as pltpu
from jax.experimental.pallas import tpu_sc as plsc

N_NODES = 10000
D_FEAT = 128
HIDDEN = 64
N_CLASSES = 40
C_PAD = 48

NC, NS = 2, 16
NW = NC * NS
B = 128
NBUF = 4
DEPTH = 3
TRASH = N_NODES
NPAD = 10112
RPT = NPAD // NS
ROWBLK = 1000


def _make_sc_agg(width, with_cnt, g_ops):
    mesh = plsc.VectorSubcoreMesh(core_axis_name="c", subcore_axis_name="s")
    acc_type = jax.ShapeDtypeStruct((NC, NPAD, width), jnp.float32)
    out_type = [acc_type]
    scratch = [
        pltpu.VMEM((g_ops, B), jnp.int32),
        pltpu.VMEM((g_ops, B), jnp.int32),
        pltpu.VMEM((2, B, width), jnp.float32),
        pltpu.VMEM_SHARED((NPAD, width), jnp.float32),
        pltpu.VMEM_SHARED((NPAD, width), jnp.float32),
        pltpu.SemaphoreType.DMA,
        pltpu.SemaphoreType.DMA((2,)),
    ]
    if with_cnt:
        out_type.append(jax.ShapeDtypeStruct((NC, NPAD, 8), jnp.float32))
        scratch += [
            pltpu.VMEM((B, 8), jnp.float32),
            pltpu.VMEM_SHARED((NPAD, 8), jnp.float32),
            pltpu.SemaphoreType.DMA,
        ]

    def body(src_hbm, dst_hbm, y_hbm, zw_hbm, *rest):
        if with_cnt:
            (z8_hbm, ones_hbm, acc_out, cnt_out,
             src_v, dst_v, rows_v, y_sh, acc_sh, gsem, ssem,
             ones_v, cnt_sh, csem) = rest
        else:
            (acc_out, src_v, dst_v, rows_v, y_sh, acc_sh, gsem, ssem) = rest
        c = lax.axis_index("c")
        s = lax.axis_index("s")
        wid = c * NS + s
        row0 = s * RPT
        pltpu.sync_copy(src_hbm.at[wid], src_v)
        pltpu.sync_copy(dst_hbm.at[wid], dst_v)
        pltpu.sync_copy(y_hbm.at[pl.ds(row0, RPT)], y_sh.at[pl.ds(row0, RPT)])
        pltpu.sync_copy(zw_hbm, acc_sh.at[pl.ds(row0, RPT)])
        if with_cnt:
            pltpu.sync_copy(ones_hbm, ones_v)
            pltpu.sync_copy(z8_hbm, cnt_sh.at[pl.ds(row0, RPT)])
        plsc.subcore_barrier()

        def issue_gather(i, b):
            pltpu.async_copy(y_sh.at[src_v.at[i]], rows_v.at[b], gsem)

        def wait_gather(b):
            pltpu.make_async_copy(y_sh.at[src_v.at[0]], rows_v.at[b],
                                  gsem).wait()

        def issue_scatter(i, b):
            pltpu.async_copy(rows_v.at[b], acc_sh.at[dst_v.at[i]],
                             ssem.at[b], add=True)
            if with_cnt:
                pltpu.async_copy(ones_v, cnt_sh.at[dst_v.at[i]], csem,
                                 add=True)

        def wait_scatter(b):
            pltpu.make_async_copy(rows_v.at[b], acc_sh.at[dst_v.at[0]],
                                  ssem.at[b]).wait()

        def wait_cnt():
            pltpu.make_async_copy(ones_v, cnt_sh.at[dst_v.at[0]],
                                  csem).wait()

        def iteration(i, b, first, last):
            wait_gather(b)
            issue_scatter(i, b)
            if not first:
                wait_scatter(1 - b)
                if with_cnt:
                    wait_cnt()
            if not last:
                issue_gather(i + 1, 1 - b)

        issue_gather(0, 0)
        iteration(0, 0, True, False)
        iteration(1, 1, False, False)

        @pl.loop(2, g_ops - 2, step=2)
        def _blk(blk):
            iteration(blk, 0, False, False)
            iteration(blk + 1, 1, False, False)

        iteration(g_ops - 2, 0, False, False)
        iteration(g_ops - 1, 1, False, True)
        wait_scatter(1)
        if with_cnt:
            wait_cnt()
        plsc.subcore_barrier()
        row0 = s * RPT
        pltpu.sync_copy(acc_sh.at[pl.ds(row0, RPT)],
                        acc_out.at[c, pl.ds(row0, RPT)])
        if with_cnt:
            pltpu.sync_copy(cnt_sh.at[pl.ds(row0, RPT)],
                            cnt_out.at[c, pl.ds(row0, RPT)])

    return pl.kernel(body, out_type=out_type if with_cnt else acc_type,
                     mesh=mesh, scratch_types=scratch,
                     compiler_params=pltpu.CompilerParams(
                         use_tc_tiling_on_sc=False))


def _t1(x_ref, w_ref, y1_ref, r1_ref):
    o = jnp.dot(x_ref[...], w_ref[...], preferred_element_type=jnp.float32)
    y1_ref[...] = o[:, :HIDDEN]
    r1_ref[...] = o[:, HIDDEN:]


def _t2(acc_ref, cnt_ref, r1_ref, b1_ref, w2l_ref, w2r_ref, b2_ref,
        y2_ref, r2_ref):
    agg = acc_ref[0] + acc_ref[1]
    cnt = jnp.maximum(cnt_ref[0, :, 0:1] + cnt_ref[1, :, 0:1], 1.0)
    h = jnp.maximum(agg / cnt + b1_ref[...] + r1_ref[...], 0.0)
    y2_ref[...] = jnp.dot(h, w2l_ref[...], preferred_element_type=jnp.float32)
    r2_ref[...] = (jnp.dot(h, w2r_ref[...], preferred_element_type=jnp.float32)
                   + b2_ref[...])


def _t3(acc2_ref, cnt_ref, r2_ref, out_ref):
    agg = acc2_ref[0] + acc2_ref[1]
    cnt = jnp.maximum(cnt_ref[0, :, 0:1] + cnt_ref[1, :, 0:1], 1.0)
    logits = agg / cnt + r2_ref[...]
    m = jnp.max(logits, axis=1, keepdims=True)
    s = jnp.sum(jnp.exp(logits - m), axis=1, keepdims=True)
    out = logits - m - jnp.log(s)
    out_ref[...] = out[:, :N_CLASSES]


_GRID = (N_NODES // ROWBLK,)

_t1_call = pl.pallas_call(
    _t1,
    grid=_GRID,
    in_specs=[
        pl.BlockSpec((ROWBLK, D_FEAT), lambda i: (i, 0)),
        pl.BlockSpec((D_FEAT, 2 * HIDDEN), lambda i: (0, 0)),
    ],
    out_specs=[
        pl.BlockSpec((ROWBLK, HIDDEN), lambda i: (i, 0)),
        pl.BlockSpec((ROWBLK, HIDDEN), lambda i: (i, 0)),
    ],
    out_shape=[jax.ShapeDtypeStruct((NPAD, HIDDEN), jnp.float32),
               jax.ShapeDtypeStruct((N_NODES, HIDDEN), jnp.float32)],
)

_t2_call = pl.pallas_call(
    _t2,
    grid=_GRID,
    in_specs=[
        pl.BlockSpec((NC, ROWBLK, HIDDEN), lambda i: (0, i, 0)),
        pl.BlockSpec((NC, ROWBLK, 8), lambda i: (0, i, 0)),
        pl.BlockSpec((ROWBLK, HIDDEN), lambda i: (i, 0)),
        pl.BlockSpec((1, HIDDEN), lambda i: (0, 0)),
        pl.BlockSpec((HIDDEN, C_PAD), lambda i: (0, 0)),
        pl.BlockSpec((HIDDEN, C_PAD), lambda i: (0, 0)),
        pl.BlockSpec((1, C_PAD), lambda i: (0, 0)),
    ],
    out_specs=[
        pl.BlockSpec((ROWBLK, C_PAD), lambda i: (i, 0)),
        pl.BlockSpec((ROWBLK, C_PAD), lambda i: (i, 0)),
    ],
    out_shape=[jax.ShapeDtypeStruct((NPAD, C_PAD), jnp.float32),
               jax.ShapeDtypeStruct((N_NODES, C_PAD), jnp.float32)],
)

_t3_call = pl.pallas_call(
    _t3,
    grid=_GRID,
    in_specs=[
        pl.BlockSpec((NC, ROWBLK, C_PAD), lambda i: (0, i, 0)),
        pl.BlockSpec((NC, ROWBLK, 8), lambda i: (0, i, 0)),
        pl.BlockSpec((ROWBLK, C_PAD), lambda i: (i, 0)),
    ],
    out_specs=pl.BlockSpec((ROWBLK, N_CLASSES), lambda i: (i, 0)),
    out_shape=jax.ShapeDtypeStruct((N_NODES, N_CLASSES), jnp.float32),
)


def kernel(x, edge_index, W1l, b1, W1r, W2l, b2, W2r):
    e = edge_index.shape[1]
    g_ops = -(-e // (NW * B))
    g_ops = -(-g_ops // NBUF) * NBUF
    e_pad = NW * B * g_ops
    src = edge_index[0].astype(jnp.int32)
    dst = edge_index[1].astype(jnp.int32)
    src = jnp.concatenate([src, jnp.zeros((e_pad - e,), jnp.int32)])
    dst = jnp.concatenate([dst, jnp.full((e_pad - e,), TRASH, jnp.int32)])
    src3 = src.reshape(NW, g_ops, B)
    dst3 = dst.reshape(NW, g_ops, B)

    zeros_h = jnp.zeros((RPT, HIDDEN), jnp.float32)
    zeros_c = jnp.zeros((RPT, C_PAD), jnp.float32)
    zeros_8 = jnp.zeros((RPT, 8), jnp.float32)
    ones_8 = jnp.ones((B, 8), jnp.float32)

    w1cat = jnp.concatenate([W1l.T, W1r.T], axis=1)
    w2l_t = jnp.zeros((HIDDEN, C_PAD), jnp.float32).at[:, :N_CLASSES].set(W2l.T)
    w2r_t = jnp.zeros((HIDDEN, C_PAD), jnp.float32).at[:, :N_CLASSES].set(W2r.T)
    b2_pad = jnp.full((1, C_PAD), -1e30, jnp.float32).at[0, :N_CLASSES].set(b2)
    b1_row = b1.reshape(1, HIDDEN)

    y1, r1 = _t1_call(x, w1cat)
    acc1, cnt = _make_sc_agg(HIDDEN, True, g_ops)(
        src3, dst3, y1, zeros_h, zeros_8, ones_8)
    y2, r2 = _t2_call(acc1, cnt, r1, b1_row, w2l_t, w2r_t, b2_pad)
    acc2 = _make_sc_agg(C_PAD, False, g_ops)(src3, dst3, y2, zeros_c)
    return _t3_call(acc2, cnt, r2)


# speedup vs baseline: 2.0234x; 1.1034x over previous
import jax
import jax.numpy as jnp
from jax import lax
from jax.experimental import pallas as pl
from jax.experimental.pallas import tpu as pltpu
from jax.experimental.pallas import tpu_sc as plsc

N_NODES = 10000
D_FEAT = 128
HIDDEN = 64
N_CLASSES = 40
C_PAD = 48

NC, NS = 2, 16
NW = NC * NS
B = 128
NBUF = 4
DEPTH = 3
TRASH = N_NODES
NPAD = 10112
RPT = NPAD // NS
ROWBLK = 1000


def _make_sc_agg(width, with_cnt, g_ops):
    mesh = plsc.VectorSubcoreMesh(core_axis_name="c", subcore_axis_name="s")
    acc_type = jax.ShapeDtypeStruct((NC, NPAD, width), jnp.float32)
    out_type = [acc_type]
    scratch = [
        pltpu.VMEM((g_ops, B), jnp.int32),
        pltpu.VMEM((g_ops, B), jnp.int32),
        pltpu.VMEM((2, B, width), jnp.float32),
        pltpu.VMEM_SHARED((NPAD, width), jnp.float32),
        pltpu.VMEM_SHARED((NPAD, width), jnp.float32),
        pltpu.SemaphoreType.DMA((2,)),
        pltpu.SemaphoreType.DMA((2,)),
    ]
    if with_cnt:
        out_type.append(jax.ShapeDtypeStruct((NC, NPAD, 8), jnp.float32))
        scratch += [
            pltpu.VMEM((B, 8), jnp.float32),
            pltpu.VMEM_SHARED((NPAD, 8), jnp.float32),
            pltpu.SemaphoreType.DMA,
        ]

    def body(src_hbm, dst_hbm, y_hbm, zw_hbm, *rest):
        if with_cnt:
            (z8_hbm, ones_hbm, acc_out, cnt_out,
             src_v, dst_v, rows_v, y_sh, acc_sh, gsem, ssem,
             ones_v, cnt_sh, csem) = rest
        else:
            (acc_out, src_v, dst_v, rows_v, y_sh, acc_sh, gsem, ssem) = rest
        c = lax.axis_index("c")
        s = lax.axis_index("s")
        wid = c * NS + s
        row0 = s * RPT
        pltpu.sync_copy(src_hbm.at[wid], src_v)
        pltpu.sync_copy(dst_hbm.at[wid], dst_v)
        pltpu.sync_copy(y_hbm.at[pl.ds(row0, RPT)], y_sh.at[pl.ds(row0, RPT)])
        pltpu.sync_copy(zw_hbm, acc_sh.at[pl.ds(row0, RPT)])
        if with_cnt:
            pltpu.sync_copy(ones_hbm, ones_v)
            pltpu.sync_copy(z8_hbm, cnt_sh.at[pl.ds(row0, RPT)])
        plsc.subcore_barrier()

        def issue_gather(i, b):
            pltpu.async_copy(y_sh.at[src_v.at[i]], rows_v.at[b], gsem.at[b])

        def wait_gather(b):
            pltpu.make_async_copy(y_sh.at[src_v.at[0]], rows_v.at[b],
                                  gsem.at[b]).wait()

        def issue_scatter(i, b):
            pltpu.async_copy(rows_v.at[b], acc_sh.at[dst_v.at[i]],
                             ssem.at[b], add=True)
            if with_cnt:
                pltpu.async_copy(ones_v, cnt_sh.at[dst_v.at[i]], csem,
                                 add=True)

        def wait_scatter(b):
            pltpu.make_async_copy(rows_v.at[b], acc_sh.at[dst_v.at[0]],
                                  ssem.at[b]).wait()

        def wait_cnt():
            pltpu.make_async_copy(ones_v, cnt_sh.at[dst_v.at[0]],
                                  csem).wait()

        def iteration(i, b, first, last):
            if not first:
                wait_scatter(1 - b)
                if with_cnt:
                    wait_cnt()
            if not last:
                issue_gather(i + 1, 1 - b)
            wait_gather(b)
            issue_scatter(i, b)

        issue_gather(0, 0)
        iteration(0, 0, True, False)
        iteration(1, 1, False, False)

        @pl.loop(2, g_ops - 2, step=2)
        def _blk(blk):
            iteration(blk, 0, False, False)
            iteration(blk + 1, 1, False, False)

        iteration(g_ops - 2, 0, False, False)
        iteration(g_ops - 1, 1, False, True)
        wait_scatter(1)
        if with_cnt:
            wait_cnt()
        plsc.subcore_barrier()
        row0 = s * RPT
        pltpu.sync_copy(acc_sh.at[pl.ds(row0, RPT)],
                        acc_out.at[c, pl.ds(row0, RPT)])
        if with_cnt:
            pltpu.sync_copy(cnt_sh.at[pl.ds(row0, RPT)],
                            cnt_out.at[c, pl.ds(row0, RPT)])

    return pl.kernel(body, out_type=out_type if with_cnt else acc_type,
                     mesh=mesh, scratch_types=scratch,
                     compiler_params=pltpu.CompilerParams(
                         use_tc_tiling_on_sc=False))


def _t1(x_ref, w_ref, y1_ref, r1_ref):
    o = jnp.dot(x_ref[...], w_ref[...], preferred_element_type=jnp.float32)
    y1_ref[...] = o[:, :HIDDEN]
    r1_ref[...] = o[:, HIDDEN:]


def _t2(acc_ref, cnt_ref, r1_ref, b1_ref, w2l_ref, w2r_ref, b2_ref,
        y2_ref, r2_ref):
    agg = acc_ref[0] + acc_ref[1]
    cnt = jnp.maximum(cnt_ref[0, :, 0:1] + cnt_ref[1, :, 0:1], 1.0)
    h = jnp.maximum(agg / cnt + b1_ref[...] + r1_ref[...], 0.0)
    y2_ref[...] = jnp.dot(h, w2l_ref[...], preferred_element_type=jnp.float32)
    r2_ref[...] = (jnp.dot(h, w2r_ref[...], preferred_element_type=jnp.float32)
                   + b2_ref[...])


def _t3(acc2_ref, cnt_ref, r2_ref, out_ref):
    agg = acc2_ref[0] + acc2_ref[1]
    cnt = jnp.maximum(cnt_ref[0, :, 0:1] + cnt_ref[1, :, 0:1], 1.0)
    logits = agg / cnt + r2_ref[...]
    m = jnp.max(logits, axis=1, keepdims=True)
    s = jnp.sum(jnp.exp(logits - m), axis=1, keepdims=True)
    out = logits - m - jnp.log(s)
    out_ref[...] = out[:, :N_CLASSES]


_GRID = (N_NODES // ROWBLK,)

_t1_call = pl.pallas_call(
    _t1,
    grid=_GRID,
    in_specs=[
        pl.BlockSpec((ROWBLK, D_FEAT), lambda i: (i, 0)),
        pl.BlockSpec((D_FEAT, 2 * HIDDEN), lambda i: (0, 0)),
    ],
    out_specs=[
        pl.BlockSpec((ROWBLK, HIDDEN), lambda i: (i, 0)),
        pl.BlockSpec((ROWBLK, HIDDEN), lambda i: (i, 0)),
    ],
    out_shape=[jax.ShapeDtypeStruct((NPAD, HIDDEN), jnp.float32),
               jax.ShapeDtypeStruct((N_NODES, HIDDEN), jnp.float32)],
)

_t2_call = pl.pallas_call(
    _t2,
    grid=_GRID,
    in_specs=[
        pl.BlockSpec((NC, ROWBLK, HIDDEN), lambda i: (0, i, 0)),
        pl.BlockSpec((NC, ROWBLK, 8), lambda i: (0, i, 0)),
        pl.BlockSpec((ROWBLK, HIDDEN), lambda i: (i, 0)),
        pl.BlockSpec((1, HIDDEN), lambda i: (0, 0)),
        pl.BlockSpec((HIDDEN, C_PAD), lambda i: (0, 0)),
        pl.BlockSpec((HIDDEN, C_PAD), lambda i: (0, 0)),
        pl.BlockSpec((1, C_PAD), lambda i: (0, 0)),
    ],
    out_specs=[
        pl.BlockSpec((ROWBLK, C_PAD), lambda i: (i, 0)),
        pl.BlockSpec((ROWBLK, C_PAD), lambda i: (i, 0)),
    ],
    out_shape=[jax.ShapeDtypeStruct((NPAD, C_PAD), jnp.float32),
               jax.ShapeDtypeStruct((N_NODES, C_PAD), jnp.float32)],
)

_t3_call = pl.pallas_call(
    _t3,
    grid=_GRID,
    in_specs=[
        pl.BlockSpec((NC, ROWBLK, C_PAD), lambda i: (0, i, 0)),
        pl.BlockSpec((NC, ROWBLK, 8), lambda i: (0, i, 0)),
        pl.BlockSpec((ROWBLK, C_PAD), lambda i: (i, 0)),
    ],
    out_specs=pl.BlockSpec((ROWBLK, N_CLASSES), lambda i: (i, 0)),
    out_shape=jax.ShapeDtypeStruct((N_NODES, N_CLASSES), jnp.float32),
)


def kernel(x, edge_index, W1l, b1, W1r, W2l, b2, W2r):
    e = edge_index.shape[1]
    g_ops = -(-e // (NW * B))
    g_ops = -(-g_ops // NBUF) * NBUF
    e_pad = NW * B * g_ops
    src = edge_index[0].astype(jnp.int32)
    dst = edge_index[1].astype(jnp.int32)
    src = jnp.concatenate([src, jnp.zeros((e_pad - e,), jnp.int32)])
    dst = jnp.concatenate([dst, jnp.full((e_pad - e,), TRASH, jnp.int32)])
    src3 = src.reshape(NW, g_ops, B)
    dst3 = dst.reshape(NW, g_ops, B)

    zeros_h = jnp.zeros((RPT, HIDDEN), jnp.float32)
    zeros_c = jnp.zeros((RPT, C_PAD), jnp.float32)
    zeros_8 = jnp.zeros((RPT, 8), jnp.float32)
    ones_8 = jnp.ones((B, 8), jnp.float32)

    w1cat = jnp.concatenate([W1l.T, W1r.T], axis=1)
    w2l_t = jnp.zeros((HIDDEN, C_PAD), jnp.float32).at[:, :N_CLASSES].set(W2l.T)
    w2r_t = jnp.zeros((HIDDEN, C_PAD), jnp.float32).at[:, :N_CLASSES].set(W2r.T)
    b2_pad = jnp.full((1, C_PAD), -1e30, jnp.float32).at[0, :N_CLASSES].set(b2)
    b1_row = b1.reshape(1, HIDDEN)

    y1, r1 = _t1_call(x, w1cat)
    acc1, cnt = _make_sc_agg(HIDDEN, True, g_ops)(
        src3, dst3, y1, zeros_h, zeros_8, ones_8)
    y2, r2 = _t2_call(acc1, cnt, r1, b1_row, w2l_t, w2r_t, b2_pad)
    acc2 = _make_sc_agg(C_PAD, False, g_ops)(src3, dst3, y2, zeros_c)
    return _t3_call(acc2, cnt, r2)


# speedup vs baseline: 2.1733x; 1.0741x over previous
import jax
import jax.numpy as jnp
from jax import lax
from jax.experimental import pallas as pl
from jax.experimental.pallas import tpu as pltpu
from jax.experimental.pallas import tpu_sc as plsc

N_NODES = 10000
D_FEAT = 128
HIDDEN = 64
N_CLASSES = 40
C_PAD = 48

NC, NS = 2, 16
NW = NC * NS
B = 128
NBUF = 4
WIN = 8
TRASH = N_NODES
NPAD = 10112
RPT = NPAD // NS
ROWBLK = 1000


def _make_sc_agg(width, with_cnt, g_ops):
    mesh = plsc.VectorSubcoreMesh(core_axis_name="c", subcore_axis_name="s")
    acc_type = jax.ShapeDtypeStruct((NC, NPAD, width), jnp.float32)
    out_type = [acc_type]
    n_win = g_ops // WIN
    scratch = [
        pltpu.VMEM((2, WIN, B), jnp.int32),
        pltpu.VMEM((2, WIN, B), jnp.int32),
        pltpu.VMEM((NBUF, B, width), jnp.float32),
        pltpu.VMEM_SHARED((NPAD, width), jnp.float32),
        pltpu.VMEM_SHARED((NPAD, width), jnp.float32),
        pltpu.SemaphoreType.DMA((2,)),
        pltpu.SemaphoreType.DMA((NBUF,)),
        pltpu.SemaphoreType.DMA((NBUF,)),
    ]
    if with_cnt:
        out_type.append(jax.ShapeDtypeStruct((NC, NPAD, 8), jnp.float32))
        scratch += [
            pltpu.VMEM((B, 8), jnp.float32),
            pltpu.VMEM_SHARED((NPAD, 8), jnp.float32),
            pltpu.SemaphoreType.DMA((NBUF,)),
        ]

    def body(src_hbm, dst_hbm, y_hbm, zw_hbm, *rest):
        if with_cnt:
            (z8_hbm, ones_hbm, acc_out, cnt_out,
             srcw, dstw, rows_v, y_sh, acc_sh, wsem, gsem, ssem,
             ones_v, cnt_sh, csem) = rest
        else:
            (acc_out, srcw, dstw, rows_v, y_sh, acc_sh,
             wsem, gsem, ssem) = rest
        c = lax.axis_index("c")
        s = lax.axis_index("s")
        wid = c * NS + s
        row0 = s * RPT
        pltpu.sync_copy(y_hbm.at[pl.ds(row0, RPT)], y_sh.at[pl.ds(row0, RPT)])
        pltpu.sync_copy(zw_hbm, acc_sh.at[pl.ds(row0, RPT)])
        if with_cnt:
            pltpu.sync_copy(ones_hbm, ones_v)
            pltpu.sync_copy(z8_hbm, cnt_sh.at[pl.ds(row0, RPT)])
        pltpu.sync_copy(src_hbm.at[wid, pl.ds(0, WIN)], srcw.at[0])
        pltpu.sync_copy(dst_hbm.at[wid, pl.ds(0, WIN)], dstw.at[0])
        plsc.subcore_barrier()

        def load_window(w, slot):
            pltpu.async_copy(src_hbm.at[wid, pl.ds(w * WIN, WIN)],
                             srcw.at[slot], wsem.at[slot])
            pltpu.async_copy(dst_hbm.at[wid, pl.ds(w * WIN, WIN)],
                             dstw.at[slot], wsem.at[slot])

        def wait_window(slot):
            pltpu.make_async_copy(src_hbm.at[wid, pl.ds(0, WIN)],
                                  srcw.at[slot], wsem.at[slot]).wait()
            pltpu.make_async_copy(dst_hbm.at[wid, pl.ds(0, WIN)],
                                  dstw.at[slot], wsem.at[slot]).wait()

        def issue_gather(slot, j, b):
            pltpu.async_copy(y_sh.at[srcw.at[slot, j]], rows_v.at[b],
                             gsem.at[b])

        def wait_gather(b):
            pltpu.make_async_copy(y_sh.at[srcw.at[0, 0]], rows_v.at[b],
                                  gsem.at[b]).wait()

        def issue_scatter(slot, j, b):
            pltpu.async_copy(rows_v.at[b], acc_sh.at[dstw.at[slot, j]],
                             ssem.at[b], add=True)
            if with_cnt:
                pltpu.async_copy(ones_v, cnt_sh.at[dstw.at[slot, j]],
                                 csem.at[b], add=True)

        def wait_scatter(b):
            pltpu.make_async_copy(rows_v.at[b], acc_sh.at[dstw.at[0, 0]],
                                  ssem.at[b]).wait()
            if with_cnt:
                pltpu.make_async_copy(ones_v, cnt_sh.at[dstw.at[0, 0]],
                                      csem.at[b]).wait()

        def window(w, s_cur, first, last):
            s_nxt = 1 - s_cur
            for j in range(WIN):
                buf = j % NBUF
                tb = (j + 2) % NBUF
                if j == 2 and not last:
                    load_window(w + 1, s_nxt)
                if j == 6 and not last:
                    wait_window(s_nxt)
                if not (first and j < 2) and not (last and j >= WIN - 2):
                    wait_scatter(tb)
                if j < WIN - 2:
                    issue_gather(s_cur, j + 2, tb)
                elif not last:
                    issue_gather(s_nxt, j - (WIN - 2), tb)
                wait_gather(buf)
                issue_scatter(s_cur, j, buf)

        issue_gather(0, 0, 0)
        issue_gather(0, 1, 1)
        window(0, 0, True, False)

        @pl.loop(0, (n_win - 2) // 2)
        def _pair(wb):
            window(2 * wb + 1, 1, False, False)
            window(2 * wb + 2, 0, False, False)

        window(n_win - 1, 1, False, True)
        for b in range(NBUF):
            wait_scatter(b)
        plsc.subcore_barrier()
        row0 = s * RPT
        pltpu.sync_copy(acc_sh.at[pl.ds(row0, RPT)],
                        acc_out.at[c, pl.ds(row0, RPT)])
        if with_cnt:
            pltpu.sync_copy(cnt_sh.at[pl.ds(row0, RPT)],
                            cnt_out.at[c, pl.ds(row0, RPT)])

    return pl.kernel(body, out_type=out_type if with_cnt else acc_type,
                     mesh=mesh, scratch_types=scratch,
                     compiler_params=pltpu.CompilerParams(
                         use_tc_tiling_on_sc=False))


def _t1(x_ref, w_ref, y1_ref, r1_ref):
    o = jnp.dot(x_ref[...], w_ref[...], preferred_element_type=jnp.float32)
    y1_ref[...] = o[:, :HIDDEN]
    r1_ref[...] = o[:, HIDDEN:]


def _t2(acc_ref, cnt_ref, r1_ref, b1_ref, w2l_ref, w2r_ref, b2_ref,
        y2_ref, r2_ref):
    agg = acc_ref[0] + acc_ref[1]
    cnt = jnp.maximum(cnt_ref[0, :, 0:1] + cnt_ref[1, :, 0:1], 1.0)
    h = jnp.maximum(agg / cnt + b1_ref[...] + r1_ref[...], 0.0)
    y2_ref[...] = jnp.dot(h, w2l_ref[...], preferred_element_type=jnp.float32)
    r2_ref[...] = (jnp.dot(h, w2r_ref[...], preferred_element_type=jnp.float32)
                   + b2_ref[...])


def _t3(acc2_ref, cnt_ref, r2_ref, out_ref):
    agg = acc2_ref[0] + acc2_ref[1]
    cnt = jnp.maximum(cnt_ref[0, :, 0:1] + cnt_ref[1, :, 0:1], 1.0)
    logits = agg / cnt + r2_ref[...]
    m = jnp.max(logits, axis=1, keepdims=True)
    s = jnp.sum(jnp.exp(logits - m), axis=1, keepdims=True)
    out = logits - m - jnp.log(s)
    out_ref[...] = out[:, :N_CLASSES]


_GRID = (N_NODES // ROWBLK,)

_t1_call = pl.pallas_call(
    _t1,
    grid=_GRID,
    in_specs=[
        pl.BlockSpec((ROWBLK, D_FEAT), lambda i: (i, 0)),
        pl.BlockSpec((D_FEAT, 2 * HIDDEN), lambda i: (0, 0)),
    ],
    out_specs=[
        pl.BlockSpec((ROWBLK, HIDDEN), lambda i: (i, 0)),
        pl.BlockSpec((ROWBLK, HIDDEN), lambda i: (i, 0)),
    ],
    out_shape=[jax.ShapeDtypeStruct((NPAD, HIDDEN), jnp.float32),
               jax.ShapeDtypeStruct((N_NODES, HIDDEN), jnp.float32)],
)

_t2_call = pl.pallas_call(
    _t2,
    grid=_GRID,
    in_specs=[
        pl.BlockSpec((NC, ROWBLK, HIDDEN), lambda i: (0, i, 0)),
        pl.BlockSpec((NC, ROWBLK, 8), lambda i: (0, i, 0)),
        pl.BlockSpec((ROWBLK, HIDDEN), lambda i: (i, 0)),
        pl.BlockSpec((1, HIDDEN), lambda i: (0, 0)),
        pl.BlockSpec((HIDDEN, C_PAD), lambda i: (0, 0)),
        pl.BlockSpec((HIDDEN, C_PAD), lambda i: (0, 0)),
        pl.BlockSpec((1, C_PAD), lambda i: (0, 0)),
    ],
    out_specs=[
        pl.BlockSpec((ROWBLK, C_PAD), lambda i: (i, 0)),
        pl.BlockSpec((ROWBLK, C_PAD), lambda i: (i, 0)),
    ],
    out_shape=[jax.ShapeDtypeStruct((NPAD, C_PAD), jnp.float32),
               jax.ShapeDtypeStruct((N_NODES, C_PAD), jnp.float32)],
)

_t3_call = pl.pallas_call(
    _t3,
    grid=_GRID,
    in_specs=[
        pl.BlockSpec((NC, ROWBLK, C_PAD), lambda i: (0, i, 0)),
        pl.BlockSpec((NC, ROWBLK, 8), lambda i: (0, i, 0)),
        pl.BlockSpec((ROWBLK, C_PAD), lambda i: (i, 0)),
    ],
    out_specs=pl.BlockSpec((ROWBLK, N_CLASSES), lambda i: (i, 0)),
    out_shape=jax.ShapeDtypeStruct((N_NODES, N_CLASSES), jnp.float32),
)


def kernel(x, edge_index, W1l, b1, W1r, W2l, b2, W2r):
    e = edge_index.shape[1]
    g_ops = -(-e // (NW * B))
    g_ops = -(-g_ops // (2 * WIN)) * (2 * WIN)
    e_pad = NW * B * g_ops
    src = edge_index[0].astype(jnp.int32)
    dst = edge_index[1].astype(jnp.int32)
    src = jnp.concatenate([src, jnp.zeros((e_pad - e,), jnp.int32)])
    dst = jnp.concatenate([dst, jnp.full((e_pad - e,), TRASH, jnp.int32)])
    src3 = src.reshape(NW, g_ops, B)
    dst3 = dst.reshape(NW, g_ops, B)

    zeros_h = jnp.zeros((RPT, HIDDEN), jnp.float32)
    zeros_c = jnp.zeros((RPT, C_PAD), jnp.float32)
    zeros_8 = jnp.zeros((RPT, 8), jnp.float32)
    ones_8 = jnp.ones((B, 8), jnp.float32)

    w1cat = jnp.concatenate([W1l.T, W1r.T], axis=1)
    w2l_t = jnp.zeros((HIDDEN, C_PAD), jnp.float32).at[:, :N_CLASSES].set(W2l.T)
    w2r_t = jnp.zeros((HIDDEN, C_PAD), jnp.float32).at[:, :N_CLASSES].set(W2r.T)
    b2_pad = jnp.full((1, C_PAD), -1e30, jnp.float32).at[0, :N_CLASSES].set(b2)
    b1_row = b1.reshape(1, HIDDEN)

    y1, r1 = _t1_call(x, w1cat)
    acc1, cnt = _make_sc_agg(HIDDEN, True, g_ops)(
        src3, dst3, y1, zeros_h, zeros_8, ones_8)
    y2, r2 = _t2_call(acc1, cnt, r1, b1_row, w2l_t, w2r_t, b2_pad)
    acc2 = _make_sc_agg(C_PAD, False, g_ops)(src3, dst3, y2, zeros_c)
    return _t3_call(acc2, cnt, r2)
